# baseline (speedup 1.0000x reference)
import functools

import jax
import jax.numpy as jnp
from jax import lax
from jax.experimental import pallas as pl
from jax.experimental.pallas import tpu as pltpu
from jax.experimental.pallas import tpu_sc as plsc

NUM_ROWS = 76
DIM = 64
B = 16384 * 200
NC, NS = 2, 16
NW = NC * NS
B_PER_W = B // NW
CHUNK = 256
NBUF = 4
N_OUTER = B_PER_W // (CHUNK * NBUF)
L = 16
GROUPS = CHUNK // L


def _emb_body(day_hbm, table_hbm, out_hbm, tab_v, idx_v, rows_v, wsems):
    wid = lax.axis_index("s") * NC + lax.axis_index("c")
    base = wid * B_PER_W

    pltpu.sync_copy(table_hbm, tab_v)

    lane = lax.iota(jnp.int32, L)

    def fill_group(b, g):
        day_vec = idx_v[pl.ds(b * CHUNK + g * L, L)]
        src_base = day_vec * DIM
        dst_base = (g * L + lane) * DIM
        for c in range(DIM):
            vals = plsc.load_gather(tab_v, [src_base + c])
            plsc.store_scatter(rows_v[b], [dst_base + c], vals)

    def chunk_step(o, b, first):
        cbase = (base + (o * NBUF + b) * CHUNK) * DIM
        if not first:
            pltpu.make_async_copy(
                rows_v[b], out_hbm.at[pl.ds(cbase, CHUNK * DIM)], wsems[b]
            ).wait()
        lax.fori_loop(0, GROUPS, lambda g, cy: (fill_group(b, g), cy)[1], 0)
        pltpu.async_copy(
            rows_v[b], out_hbm.at[pl.ds(cbase, CHUNK * DIM)], wsems[b]
        )

    def load_slab(o):
        pltpu.sync_copy(
            day_hbm.at[pl.ds(base + o * (NBUF * CHUNK), NBUF * CHUNK)], idx_v
        )

    load_slab(0)
    for b in range(NBUF):
        chunk_step(0, b, True)

    def outer(o, carry):
        load_slab(o)
        for b in range(NBUF):
            chunk_step(o, b, False)
        return carry

    lax.fori_loop(1, N_OUTER, outer, 0)

    for b in range(NBUF):
        cbase = (base + ((N_OUTER - 1) * NBUF + b) * CHUNK) * DIM
        pltpu.make_async_copy(
            rows_v[b], out_hbm.at[pl.ds(cbase, CHUNK * DIM)], wsems[b]
        ).wait()


@jax.jit
def _emb(day_flat, table_flat):
    mesh = plsc.VectorSubcoreMesh(core_axis_name="c", subcore_axis_name="s")
    f = functools.partial(
        pl.kernel,
        out_type=jax.ShapeDtypeStruct((B * DIM,), jnp.float32),
        mesh=mesh,
        scratch_types=[
            pltpu.VMEM((NUM_ROWS * DIM,), jnp.float32),
            pltpu.VMEM((NBUF * CHUNK,), jnp.int32),
            [pltpu.VMEM((CHUNK * DIM,), jnp.float32)] * NBUF,
            [pltpu.SemaphoreType.DMA] * NBUF,
        ],
        compiler_params=pltpu.CompilerParams(needs_layout_passes=False),
    )(_emb_body)
    return f(day_flat, table_flat)


def kernel(day, table):
    day_flat = day.reshape(B)
    out = _emb(day_flat, table.reshape(NUM_ROWS * DIM))
    return out.reshape(day.shape[0], day.shape[1], DIM)


# speedup vs baseline: 1.9658x; 1.9658x over previous
import functools

import jax
import jax.numpy as jnp
from jax import lax
from jax.experimental import pallas as pl
from jax.experimental.pallas import tpu as pltpu
from jax.experimental.pallas import tpu_sc as plsc

NUM_ROWS = 76
DIM = 64
B = 16384 * 200
NC, NS = 2, 16
NW = NC * NS
B_PER_W = B // NW
CHUNK = 256
NBUF = 4
N_OUTER = B_PER_W // (CHUNK * NBUF)


def _emb_body(day_hbm, table_hbm, out_hbm, idx_v, rows_v, gsems, wsems):
    wid = lax.axis_index("s") * NC + lax.axis_index("c")
    base = wid * B_PER_W

    def chunk_step(o, b, first):
        cbase = base + (o * NBUF + b) * CHUNK
        if not first:
            pltpu.make_async_copy(
                rows_v[b], out_hbm.at[pl.ds(cbase, CHUNK)], wsems[b]
            ).wait()
        pltpu.async_copy(
            table_hbm.at[idx_v.at[pl.ds(b * CHUNK, CHUNK)]],
            rows_v[b],
            gsems[b],
        ).wait()
        pltpu.async_copy(
            rows_v[b], out_hbm.at[pl.ds(cbase, CHUNK)], wsems[b]
        )

    def load_slab(o):
        pltpu.sync_copy(
            day_hbm.at[pl.ds(base + o * (NBUF * CHUNK), NBUF * CHUNK)], idx_v
        )

    load_slab(0)
    for b in range(NBUF):
        chunk_step(0, b, True)

    def outer(o, carry):
        load_slab(o)
        for b in range(NBUF):
            chunk_step(o, b, False)
        return carry

    lax.fori_loop(1, N_OUTER, outer, 0)

    for b in range(NBUF):
        cbase = base + ((N_OUTER - 1) * NBUF + b) * CHUNK
        pltpu.make_async_copy(
            rows_v[b], out_hbm.at[pl.ds(cbase, CHUNK)], wsems[b]
        ).wait()


@jax.jit
def _emb(day_flat, table):
    mesh = plsc.VectorSubcoreMesh(core_axis_name="c", subcore_axis_name="s")
    f = functools.partial(
        pl.kernel,
        out_type=jax.ShapeDtypeStruct((B, DIM), jnp.float32),
        mesh=mesh,
        scratch_types=[
            pltpu.VMEM((NBUF * CHUNK,), jnp.int32),
            [pltpu.VMEM((CHUNK, DIM), jnp.float32)] * NBUF,
            [pltpu.SemaphoreType.DMA] * NBUF,
            [pltpu.SemaphoreType.DMA] * NBUF,
        ],
        compiler_params=pltpu.CompilerParams(use_tc_tiling_on_sc=False),
    )(_emb_body)
    return f(day_flat, table)


def kernel(day, table):
    day_flat = day.reshape(B)
    out = _emb(day_flat, table)
    return out.reshape(day.shape[0], day.shape[1], DIM)


# speedup vs baseline: 4.0129x; 2.0414x over previous
import functools

import jax
import jax.numpy as jnp
from jax import lax
from jax.experimental import pallas as pl
from jax.experimental.pallas import tpu as pltpu
from jax.experimental.pallas import tpu_sc as plsc

NUM_ROWS = 76
DIM = 64
B = 16384 * 200
NC, NS = 2, 16
NW = NC * NS
B_PER_W = B // NW
CHUNK = 256
NBUF = 4
N_OUTER = B_PER_W // (CHUNK * NBUF)


def _emb_body(day_hbm, table_hbm, out_hbm, tab_v, idx_v, rows_v, gsems, wsems):
    sid = lax.axis_index("s")
    wid = sid * NC + lax.axis_index("c")
    base = wid * B_PER_W

    @pl.when(sid == 0)
    def _():
        pltpu.sync_copy(table_hbm, tab_v)

    plsc.subcore_barrier()

    def chunk_step(o, b, first):
        cbase = base + (o * NBUF + b) * CHUNK
        if not first:
            pltpu.make_async_copy(
                rows_v[b], out_hbm.at[pl.ds(cbase, CHUNK)], wsems[b]
            ).wait()
        pltpu.async_copy(
            tab_v.at[idx_v.at[pl.ds(b * CHUNK, CHUNK)]],
            rows_v[b],
            gsems[b],
        ).wait()
        pltpu.async_copy(
            rows_v[b], out_hbm.at[pl.ds(cbase, CHUNK)], wsems[b]
        )

    def load_slab(o):
        pltpu.sync_copy(
            day_hbm.at[pl.ds(base + o * (NBUF * CHUNK), NBUF * CHUNK)], idx_v
        )

    load_slab(0)
    for b in range(NBUF):
        chunk_step(0, b, True)

    def outer(o, carry):
        load_slab(o)
        for b in range(NBUF):
            chunk_step(o, b, False)
        return carry

    lax.fori_loop(1, N_OUTER, outer, 0)

    for b in range(NBUF):
        cbase = base + ((N_OUTER - 1) * NBUF + b) * CHUNK
        pltpu.make_async_copy(
            rows_v[b], out_hbm.at[pl.ds(cbase, CHUNK)], wsems[b]
        ).wait()


@jax.jit
def _emb(day_flat, table):
    mesh = plsc.VectorSubcoreMesh(core_axis_name="c", subcore_axis_name="s")
    f = functools.partial(
        pl.kernel,
        out_type=jax.ShapeDtypeStruct((B, DIM), jnp.float32),
        mesh=mesh,
        scratch_types=[
            pltpu.VMEM_SHARED((NUM_ROWS, DIM), jnp.float32),
            pltpu.VMEM((NBUF * CHUNK,), jnp.int32),
            [pltpu.VMEM((CHUNK, DIM), jnp.float32)] * NBUF,
            [pltpu.SemaphoreType.DMA] * NBUF,
            [pltpu.SemaphoreType.DMA] * NBUF,
        ],
        compiler_params=pltpu.CompilerParams(use_tc_tiling_on_sc=False),
    )(_emb_body)
    return f(day_flat, table)


def kernel(day, table):
    day_flat = day.reshape(B)
    out = _emb(day_flat, table)
    return out.reshape(day.shape[0], day.shape[1], DIM)


# speedup vs baseline: 9.2897x; 2.3150x over previous
import functools

import jax
import jax.numpy as jnp
from jax import lax
from jax.experimental import pallas as pl
from jax.experimental.pallas import tpu as pltpu
from jax.experimental.pallas import tpu_sc as plsc

NUM_ROWS = 76
DIM = 64
NSEQ = 16384
TSTEP = 200
B = NSEQ * TSTEP
P = B // 2
NC, NS = 2, 16
NW = NC * NS
P_PER_W = P // NW
PCHUNK = 128
NBUF = 4
N_OUTER = P_PER_W // (PCHUNK * NBUF)
L = 16
SLAB = NBUF * PCHUNK
SLAB_GROUPS = SLAB // L

KN = 128
PAIRS_PER_BLOCK = KN * TSTEP // 2
N_BLOCKS = NSEQ // KN


def _gather_body(day_hbm, pt_hbm, g_hbm, tab_v, idx_v, pidx_v, rows_v,
                 gsems, wsems):
    sid = lax.axis_index("s")
    wid = sid * NC + lax.axis_index("c")
    base = wid * P_PER_W

    @pl.when(sid == 0)
    def _():
        pltpu.sync_copy(pt_hbm, tab_v)

    plsc.subcore_barrier()

    lane2 = lax.iota(jnp.int32, L) * 2

    def load_slab(o):
        pltpu.sync_copy(
            day_hbm.at[pl.ds((base + o * SLAB) * 2, SLAB * 2)], idx_v
        )
        for g in range(SLAB_GROUPS):
            ev = plsc.load_gather(idx_v, [lane2 + (g * 2 * L)])
            od = plsc.load_gather(idx_v, [lane2 + (g * 2 * L + 1)])
            pidx_v[pl.ds(g * L, L)] = ev * NUM_ROWS + od

    def chunk_step(o, b, first):
        cbase = base + (o * NBUF + b) * PCHUNK
        if not first:
            pltpu.make_async_copy(
                rows_v[b], g_hbm.at[pl.ds(cbase, PCHUNK)], wsems[b]
            ).wait()
        pltpu.async_copy(
            tab_v.at[pidx_v.at[pl.ds(b * PCHUNK, PCHUNK)]],
            rows_v[b],
            gsems[b],
        ).wait()
        pltpu.async_copy(
            rows_v[b], g_hbm.at[pl.ds(cbase, PCHUNK)], wsems[b]
        )

    load_slab(0)
    for b in range(NBUF):
        chunk_step(0, b, True)

    def outer(o, carry):
        load_slab(o)
        for b in range(NBUF):
            chunk_step(o, b, False)
        return carry

    lax.fori_loop(1, N_OUTER, outer, 0)

    for b in range(NBUF):
        cbase = base + ((N_OUTER - 1) * NBUF + b) * PCHUNK
        pltpu.make_async_copy(
            rows_v[b], g_hbm.at[pl.ds(cbase, PCHUNK)], wsems[b]
        ).wait()


@jax.jit
def _emb(day_flat, pair_table):
    mesh = plsc.VectorSubcoreMesh(core_axis_name="c", subcore_axis_name="s")
    f = functools.partial(
        pl.kernel,
        out_type=jax.ShapeDtypeStruct((P, 2 * DIM), jnp.float32),
        mesh=mesh,
        scratch_types=[
            pltpu.VMEM_SHARED((NUM_ROWS * NUM_ROWS, 2 * DIM), jnp.float32),
            pltpu.VMEM((SLAB * 2,), jnp.int32),
            pltpu.VMEM((SLAB,), jnp.int32),
            [pltpu.VMEM((PCHUNK, 2 * DIM), jnp.float32)] * NBUF,
            [pltpu.SemaphoreType.DMA] * NBUF,
            [pltpu.SemaphoreType.DMA] * NBUF,
        ],
        compiler_params=pltpu.CompilerParams(needs_layout_passes=False),
    )(_gather_body)
    return f(day_flat, pair_table)


def _tp_body(g_ref, o_ref):
    x = g_ref[...].reshape(KN, PAIRS_PER_BLOCK * 2 * DIM // KN)
    o_ref[...] = x.T.reshape(TSTEP, DIM, KN)


@jax.jit
def _transpose_tc(g):
    return pl.pallas_call(
        _tp_body,
        grid=(N_BLOCKS,),
        in_specs=[
            pl.BlockSpec((PAIRS_PER_BLOCK, 2 * DIM), lambda i: (i, 0)),
        ],
        out_specs=pl.BlockSpec((TSTEP, DIM, KN), lambda i: (0, 0, i)),
        out_shape=jax.ShapeDtypeStruct((TSTEP, DIM, NSEQ), jnp.float32),
    )(g)


def kernel(day, table):
    day_flat = day.reshape(B)
    pair_table = jnp.concatenate(
        [
            jnp.repeat(table, NUM_ROWS, axis=0),
            jnp.tile(table, (NUM_ROWS, 1)),
        ],
        axis=1,
    )
    g = _emb(day_flat, pair_table)
    out_p = _transpose_tc(g)
    return out_p.transpose(2, 0, 1)


# speedup vs baseline: 10.2436x; 1.1027x over previous
import functools

import jax
import jax.numpy as jnp
from jax import lax
from jax.experimental import pallas as pl
from jax.experimental.pallas import tpu as pltpu
from jax.experimental.pallas import tpu_sc as plsc

NUM_ROWS = 76
DIM = 64
NSEQ = 16384
TSTEP = 200
B = NSEQ * TSTEP
P = B // 2
NC, NS = 2, 16
NW = NC * NS
NCH = 4
P_CH = P // NCH
P_PER_W = P_CH // NW
PCHUNK = 128
NBUF = 4
N_OUTER = P_PER_W // (PCHUNK * NBUF)
L = 16
SLAB = NBUF * PCHUNK
SLAB_GROUPS = SLAB // L

KN = 128
PAIRS_PER_BLOCK = KN * TSTEP // 2
N_BLOCKS = P_CH // PAIRS_PER_BLOCK


def _make_gather_body(chunk):
    def _gather_body(day_hbm, pt_hbm, g_hbm, tab_v, idx_v, pidx_v, rows_v,
                     gsems, wsems):
        sid = lax.axis_index("s")
        wid = sid * NC + lax.axis_index("c")
        base = chunk * P_CH + wid * P_PER_W

        @pl.when(sid == 0)
        def _():
            pltpu.sync_copy(pt_hbm, tab_v)

        plsc.subcore_barrier()

        lane2 = lax.iota(jnp.int32, L) * 2

        def load_slab(o):
            pltpu.sync_copy(
                day_hbm.at[pl.ds((base + o * SLAB) * 2, SLAB * 2)], idx_v
            )
            for g in range(SLAB_GROUPS):
                ev = plsc.load_gather(idx_v, [lane2 + (g * 2 * L)])
                od = plsc.load_gather(idx_v, [lane2 + (g * 2 * L + 1)])
                pidx_v[pl.ds(g * L, L)] = ev * NUM_ROWS + od

        def chunk_step(o, b, first):
            cbase = (wid * P_PER_W) + (o * NBUF + b) * PCHUNK
            if not first:
                pltpu.make_async_copy(
                    rows_v[b], g_hbm.at[pl.ds(cbase, PCHUNK)], wsems[b]
                ).wait()
            pltpu.async_copy(
                tab_v.at[pidx_v.at[pl.ds(b * PCHUNK, PCHUNK)]],
                rows_v[b],
                gsems[b],
            ).wait()
            pltpu.async_copy(
                rows_v[b], g_hbm.at[pl.ds(cbase, PCHUNK)], wsems[b]
            )

        load_slab(0)
        for b in range(NBUF):
            chunk_step(0, b, True)

        def outer(o, carry):
            load_slab(o)
            for b in range(NBUF):
                chunk_step(o, b, False)
            return carry

        lax.fori_loop(1, N_OUTER, outer, 0)

        for b in range(NBUF):
            cbase = (wid * P_PER_W) + ((N_OUTER - 1) * NBUF + b) * PCHUNK
            pltpu.make_async_copy(
                rows_v[b], g_hbm.at[pl.ds(cbase, PCHUNK)], wsems[b]
            ).wait()

    return _gather_body


def _sc_gather(chunk, day_flat, pair_table):
    mesh = plsc.VectorSubcoreMesh(core_axis_name="c", subcore_axis_name="s")
    f = functools.partial(
        pl.kernel,
        out_type=jax.ShapeDtypeStruct((P_CH, 2 * DIM), jnp.float32),
        mesh=mesh,
        scratch_types=[
            pltpu.VMEM_SHARED((NUM_ROWS * NUM_ROWS, 2 * DIM), jnp.float32),
            pltpu.VMEM((SLAB * 2,), jnp.int32),
            pltpu.VMEM((SLAB,), jnp.int32),
            [pltpu.VMEM((PCHUNK, 2 * DIM), jnp.float32)] * NBUF,
            [pltpu.SemaphoreType.DMA] * NBUF,
            [pltpu.SemaphoreType.DMA] * NBUF,
        ],
        compiler_params=pltpu.CompilerParams(needs_layout_passes=False),
    )(_make_gather_body(chunk))
    return f(day_flat, pair_table)


def _tp_body_first(g_ref, o_ref):
    x = g_ref[...].reshape(KN, PAIRS_PER_BLOCK * 2 * DIM // KN)
    o_ref[...] = x.T.reshape(TSTEP, DIM, KN)


def _tp_body_chained(acc_ref, g_ref, o_ref):
    x = g_ref[...].reshape(KN, PAIRS_PER_BLOCK * 2 * DIM // KN)
    o_ref[...] = x.T.reshape(TSTEP, DIM, KN)


def _tc_transpose(chunk, g, acc):
    out_shape = jax.ShapeDtypeStruct((TSTEP, DIM, NSEQ), jnp.float32)
    g_spec = pl.BlockSpec((PAIRS_PER_BLOCK, 2 * DIM), lambda i: (i, 0))
    o_spec = pl.BlockSpec(
        (TSTEP, DIM, KN), lambda i, c=chunk: (0, 0, c * N_BLOCKS + i)
    )
    if acc is None:
        return pl.pallas_call(
            _tp_body_first,
            grid=(N_BLOCKS,),
            in_specs=[g_spec],
            out_specs=o_spec,
            out_shape=out_shape,
        )(g)
    return pl.pallas_call(
        _tp_body_chained,
        grid=(N_BLOCKS,),
        in_specs=[pl.BlockSpec(memory_space=pl.ANY), g_spec],
        out_specs=o_spec,
        out_shape=out_shape,
        input_output_aliases={0: 0},
    )(acc, g)


@jax.jit
def _emb_pipeline(day_flat, pair_table):
    gs = [_sc_gather(c, day_flat, pair_table) for c in range(NCH)]
    acc = None
    for c in range(NCH):
        acc = _tc_transpose(c, gs[c], acc)
    return acc


def kernel(day, table):
    day_flat = day.reshape(B)
    pair_table = jnp.concatenate(
        [
            jnp.repeat(table, NUM_ROWS, axis=0),
            jnp.tile(table, (NUM_ROWS, 1)),
        ],
        axis=1,
    )
    out_p = _emb_pipeline(day_flat, pair_table)
    return out_p.transpose(2, 0, 1)


# speedup vs baseline: 10.2678x; 1.0024x over previous
import functools

import jax
import jax.numpy as jnp
from jax import lax
from jax.experimental import pallas as pl
from jax.experimental.pallas import tpu as pltpu
from jax.experimental.pallas import tpu_sc as plsc

NUM_ROWS = 76
DIM = 64
NSEQ = 16384
TSTEP = 200
B = NSEQ * TSTEP
P = B // 2
NC, NS = 2, 16
NW = NC * NS
NCH = 8
P_CH = P // NCH
P_PER_W = P_CH // NW
PCHUNK = 128
NBUF = 2
N_OUTER = P_PER_W // (PCHUNK * NBUF)
L = 16
SLAB = NBUF * PCHUNK
SLAB_GROUPS = SLAB // L

KN = 128
PAIRS_PER_BLOCK = KN * TSTEP // 2
N_BLOCKS = P_CH // PAIRS_PER_BLOCK


def _make_gather_body(chunk):
    def _gather_body(day_hbm, pt_hbm, g_hbm, tab_v, idx_v, pidx_v, rows_v,
                     gsems, wsems):
        sid = lax.axis_index("s")
        wid = sid * NC + lax.axis_index("c")
        base = chunk * P_CH + wid * P_PER_W

        @pl.when(sid == 0)
        def _():
            pltpu.sync_copy(pt_hbm, tab_v)

        plsc.subcore_barrier()

        lane2 = lax.iota(jnp.int32, L) * 2

        def load_slab(o):
            pltpu.sync_copy(
                day_hbm.at[pl.ds((base + o * SLAB) * 2, SLAB * 2)], idx_v
            )
            for g in range(SLAB_GROUPS):
                ev = plsc.load_gather(idx_v, [lane2 + (g * 2 * L)])
                od = plsc.load_gather(idx_v, [lane2 + (g * 2 * L + 1)])
                pidx_v[pl.ds(g * L, L)] = ev * NUM_ROWS + od

        def chunk_step(o, b, first):
            cbase = (wid * P_PER_W) + (o * NBUF + b) * PCHUNK
            if not first:
                pltpu.make_async_copy(
                    rows_v[b], g_hbm.at[pl.ds(cbase, PCHUNK)], wsems[b]
                ).wait()
            pltpu.async_copy(
                tab_v.at[pidx_v.at[pl.ds(b * PCHUNK, PCHUNK)]],
                rows_v[b],
                gsems[b],
            ).wait()
            pltpu.async_copy(
                rows_v[b], g_hbm.at[pl.ds(cbase, PCHUNK)], wsems[b]
            )

        load_slab(0)
        for b in range(NBUF):
            chunk_step(0, b, True)

        def outer(o, carry):
            load_slab(o)
            for b in range(NBUF):
                chunk_step(o, b, False)
            return carry

        lax.fori_loop(1, N_OUTER, outer, 0)

        for b in range(NBUF):
            cbase = (wid * P_PER_W) + ((N_OUTER - 1) * NBUF + b) * PCHUNK
            pltpu.make_async_copy(
                rows_v[b], g_hbm.at[pl.ds(cbase, PCHUNK)], wsems[b]
            ).wait()

    return _gather_body


def _sc_gather(chunk, day_flat, pair_table):
    mesh = plsc.VectorSubcoreMesh(core_axis_name="c", subcore_axis_name="s")
    f = functools.partial(
        pl.kernel,
        out_type=jax.ShapeDtypeStruct((P_CH, 2 * DIM), jnp.float32),
        mesh=mesh,
        scratch_types=[
            pltpu.VMEM_SHARED((NUM_ROWS * NUM_ROWS, 2 * DIM), jnp.float32),
            pltpu.VMEM((SLAB * 2,), jnp.int32),
            pltpu.VMEM((SLAB,), jnp.int32),
            [pltpu.VMEM((PCHUNK, 2 * DIM), jnp.float32)] * NBUF,
            [pltpu.SemaphoreType.DMA] * NBUF,
            [pltpu.SemaphoreType.DMA] * NBUF,
        ],
        compiler_params=pltpu.CompilerParams(needs_layout_passes=False),
    )(_make_gather_body(chunk))
    return f(day_flat, pair_table)


def _tp_body_first(g_ref, o_ref):
    x = g_ref[...].reshape(KN, PAIRS_PER_BLOCK * 2 * DIM // KN)
    o_ref[...] = x.T.reshape(TSTEP, DIM, KN)


def _tp_body_chained(acc_ref, g_ref, o_ref):
    x = g_ref[...].reshape(KN, PAIRS_PER_BLOCK * 2 * DIM // KN)
    o_ref[...] = x.T.reshape(TSTEP, DIM, KN)


def _tc_transpose(chunk, g, acc):
    out_shape = jax.ShapeDtypeStruct((TSTEP, DIM, NSEQ), jnp.float32)
    g_spec = pl.BlockSpec((PAIRS_PER_BLOCK, 2 * DIM), lambda i: (i, 0))
    o_spec = pl.BlockSpec(
        (TSTEP, DIM, KN), lambda i, c=chunk: (0, 0, c * N_BLOCKS + i)
    )
    if acc is None:
        return pl.pallas_call(
            _tp_body_first,
            grid=(N_BLOCKS,),
            in_specs=[g_spec],
            out_specs=o_spec,
            out_shape=out_shape,
        )(g)
    return pl.pallas_call(
        _tp_body_chained,
        grid=(N_BLOCKS,),
        in_specs=[pl.BlockSpec(memory_space=pl.ANY), g_spec],
        out_specs=o_spec,
        out_shape=out_shape,
        input_output_aliases={0: 0},
    )(acc, g)


@jax.jit
def _emb_pipeline(day_flat, pair_table):
    gs = [_sc_gather(c, day_flat, pair_table) for c in range(NCH)]
    acc = None
    for c in range(NCH):
        acc = _tc_transpose(c, gs[c], acc)
    return acc


def kernel(day, table):
    day_flat = day.reshape(B)
    pair_table = jnp.concatenate(
        [
            jnp.repeat(table, NUM_ROWS, axis=0),
            jnp.tile(table, (NUM_ROWS, 1)),
        ],
        axis=1,
    )
    out_p = _emb_pipeline(day_flat, pair_table)
    return out_p.transpose(2, 0, 1)
